# baseline (speedup 1.0000x reference)
import functools

import jax
import jax.numpy as jnp
from jax import lax
from jax.experimental import pallas as pl
from jax.experimental.pallas import tpu as pltpu
from jax.experimental.pallas import tpu_sc as plsc

N_EMBED = 1_000_000
D_MODEL = 64
B_TOTAL = 16384 * 50
NC, NS = 2, 16
NW = NC * NS
B_PER_W = B_TOTAL // NW
CHUNK = 128
NCHUNK = B_PER_W // CHUNK
NBUF = 8
DEPTH = 4

_MESH = plsc.VectorSubcoreMesh(
    core_axis_name="c", subcore_axis_name="s", num_cores=NC, num_subcores=NS
)


@functools.partial(
    pl.kernel,
    out_type=jax.ShapeDtypeStruct((B_TOTAL, D_MODEL), jnp.float32),
    mesh=_MESH,
    scratch_types=[
        pltpu.VMEM((NCHUNK, CHUNK), jnp.int32),
        pltpu.VMEM((NBUF, CHUNK, D_MODEL), jnp.float32),
    ]
    + [pltpu.SemaphoreType.DMA] * (2 * NBUF),
    compiler_params=pltpu.CompilerParams(use_tc_tiling_on_sc=False),
)
def _embed_gather(idx_hbm, table_hbm, out_hbm, idx_v, rows_v, *sems):
    g_sems, s_sems = sems[:NBUF], sems[NBUF:]
    wid = lax.axis_index("s") * NC + lax.axis_index("c")
    base = wid * B_PER_W

    pltpu.sync_copy(idx_hbm.at[wid], idx_v)

    def fire_gather(g, b):
        pltpu.async_copy(table_hbm.at[idx_v.at[g]], rows_v.at[b], g_sems[b])

    def drain_gather(b):
        pltpu.make_async_copy(
            table_hbm.at[idx_v.at[0]], rows_v.at[b], g_sems[b]
        ).wait()

    def fire_store(g, b):
        pltpu.async_copy(
            rows_v.at[b], out_hbm.at[pl.ds(base + g * CHUNK, CHUNK)], s_sems[b]
        )

    def drain_store(b):
        pltpu.make_async_copy(
            rows_v.at[b], out_hbm.at[pl.ds(base, CHUNK)], s_sems[b]
        ).wait()

    def step(g, b, do_store_drain, do_fire):
        drain_gather(b)
        fire_store(g, b)
        if do_store_drain:
            drain_store((b + DEPTH) % NBUF)
        if do_fire:
            fire_gather(g + DEPTH, (b + DEPTH) % NBUF)

    for g in range(DEPTH):
        fire_gather(g, g)
    for g in range(DEPTH):
        step(g, g, False, True)

    @pl.loop(DEPTH, NCHUNK - DEPTH, step=NBUF)
    def _(t):
        for db in range(NBUF):
            step(t + db, (DEPTH + db) % NBUF, True, True)

    for g in range(NCHUNK - DEPTH, NCHUNK):
        step(g, g % NBUF, True, False)
    for g in range(NCHUNK - DEPTH, NCHUNK):
        drain_store(g % NBUF)


def kernel(x, weight):
    idx = x.reshape(-1).astype(jnp.int32).reshape(NW, NCHUNK, CHUNK)
    out = _embed_gather(idx, weight)
    return out.reshape(x.shape + (D_MODEL,))
